# baseline (speedup 1.0000x reference)
import jax
import jax.numpy as jnp
from jax.experimental import pallas as pl
from jax.experimental.pallas import tpu as pltpu
from jax.experimental.pallas import tpu_sc as plsc


def kernel(input, weights):
    n, dim = weights.shape
    seq_len = input.shape[1]
    nsub = 32
    rows_per = seq_len // nsub
    chunk = 16

    mesh = plsc.VectorSubcoreMesh(core_axis_name="core", subcore_axis_name="subcore")

    @pl.kernel(
        out_type=jax.ShapeDtypeStruct((seq_len, dim), jnp.float32),
        mesh=mesh,
        scratch_types=[pltpu.VMEM((chunk, dim), jnp.float32)],
    )
    def sc_copy(x_hbm, o_hbm, tmp):
        c = jax.lax.axis_index("core")
        s = jax.lax.axis_index("subcore")
        sid = c * 16 + s
        out_start = sid * rows_per
        in_start = (out_start % n)

        @pl.loop(0, rows_per, step=chunk)
        def _(r):
            pltpu.sync_copy(x_hbm.at[pl.ds(in_start + r, chunk)], tmp)
            pltpu.sync_copy(tmp, o_hbm.at[pl.ds(out_start + r, chunk)])

    return jax.lax.stop_gradient(sc_copy(weights))


# speedup vs baseline: 3.2104x; 3.2104x over previous
import functools
import math

import jax
import jax.numpy as jnp
from jax.experimental import pallas as pl
from jax.experimental.pallas import tpu as pltpu


def _body(w_ref, o_ref, c_ref, s_ref, *, rows, scale, half, nsteps):
    i = pl.program_id(0)

    @pl.when(i == 0)
    def _():
        k = jax.lax.broadcasted_iota(jnp.int32, (nsteps, half), 0).astype(jnp.float32)
        j = jax.lax.broadcasted_iota(jnp.int32, (nsteps, half), 1).astype(jnp.float32)
        ang = (k * float(rows)) * jnp.exp(j * (-scale))
        c_ref[...] = jnp.cos(ang)
        s_ref[...] = jnp.sin(ang)

    w = w_ref[...]
    ws = w[:, :half]
    wc = w[:, half:]
    c = c_ref[pl.ds(i, 1), :]
    s = s_ref[pl.ds(i, 1), :]
    o_ref[:, :half] = ws * c + wc * s
    o_ref[:, half:] = wc * c - ws * s


def kernel(input, weights):
    _, dim = weights.shape
    half = dim // 2
    seq_len = input.shape[1]
    scale = math.log(10000.0) / (half - 1)
    rows = 256
    nsteps = seq_len // rows
    out = pl.pallas_call(
        functools.partial(_body, rows=rows, scale=scale, half=half, nsteps=nsteps),
        grid=(nsteps,),
        in_specs=[pl.BlockSpec((rows, dim), lambda i: (0, 0))],
        out_specs=pl.BlockSpec((rows, dim), lambda i: (i, 0)),
        out_shape=jax.ShapeDtypeStruct((seq_len, dim), jnp.float32),
        scratch_shapes=[
            pltpu.VMEM((nsteps, half), jnp.float32),
            pltpu.VMEM((nsteps, half), jnp.float32),
        ],
    )(weights)
    return jax.lax.stop_gradient(out)
